# baseline (speedup 1.0000x reference)
import functools

import jax
import jax.numpy as jnp
from jax import lax
from jax.experimental import pallas as pl
from jax.experimental.pallas import tpu as pltpu
from jax.experimental.pallas import tpu_sc as plsc

N = 10000
NP = 10240
D = 128
NCLS = 10
E = 160000
NC = 2
NS = 16
NW = NC * NS
IW = 128
EROWS = E // IW
ROWS_PER_TILE = 40
EPT = ROWS_PER_TILE * IW
CW = 64
NPAIR = EPT // (2 * CW)
STRIPE = NP // NS
NB = NP // 512
BLK = 512

_MESH = plsc.VectorSubcoreMesh(
    core_axis_name="c", subcore_axis_name="s", num_cores=NC, num_subcores=NS)


def _wid():
    return lax.axis_index("s") * NC + lax.axis_index("c")


def _row_range(wid):
    start = wid * ROWS_PER_TILE
    end = jnp.minimum(start + ROWS_PER_TILE, EROWS)
    return start, end


@functools.partial(
    pl.kernel,
    out_type=jax.ShapeDtypeStruct((6 * NW * NP,), jnp.float32),
    mesh=_MESH,
    scratch_types=[
        [pltpu.VMEM((NP,), jnp.float32)] * 6,
        pltpu.VMEM((EPT,), jnp.int32),
    ],
    compiler_params=pltpu.CompilerParams(needs_layout_passes=False),
)
def _sc_degrees(s0, d0, s1, d1, s2, d2, degpart, deg_vs, idx_v):
    wid = _wid()
    ones = jnp.full((16,), 1.0, jnp.float32)
    base = wid * EPT
    ngroups = jnp.where(wid == NW - 1, (E - (NW - 1) * EPT) // 16, EPT // 16)

    for deg_v in deg_vs:
        def zbody(i, _, deg_v=deg_v):
            deg_v[pl.ds(i * 16, 16)] = jnp.zeros((16,), jnp.float32)
            return 0
        lax.fori_loop(0, NP // 16, zbody, 0)

    for a, idx_hbm in enumerate((s0, d0, s1, d1, s2, d2)):
        @pl.when(wid < NW - 1)
        def _(idx_hbm=idx_hbm):
            pltpu.sync_copy(idx_hbm.at[pl.ds(base, EPT)], idx_v)

        @pl.when(wid == NW - 1)
        def _(idx_hbm=idx_hbm):
            tail = E - (NW - 1) * EPT
            pltpu.sync_copy(idx_hbm.at[pl.ds(base, tail)],
                            idx_v.at[pl.ds(0, tail)])

        def gbody(g, _, a=a):
            iv = idx_v[pl.ds(g * 16, 16)]
            plsc.addupdate_scatter(deg_vs[a], [iv], ones)
            return 0
        lax.fori_loop(0, ngroups, gbody, 0)

    for a in range(6):
        pltpu.sync_copy(deg_vs[a], degpart.at[pl.ds((a * NW + wid) * NP, NP)])


def _tc_prep_body(degpart_ref, x_ref, w1_ref, n0_ref, p0_ref, p1_ref, p2_ref):
    deg = jnp.sum(degpart_ref[...], axis=1)
    norm = jnp.where(deg > 0.0, lax.rsqrt(deg), 0.0)
    n0_ref[...] = norm
    x_blk = x_ref[...]
    for r, p_ref in enumerate((p0_ref, p1_ref, p2_ref)):
        xw = jnp.dot(x_blk, w1_ref[r], preferred_element_type=jnp.float32)
        p_ref[...] = xw * norm[2 * r][:, None]


def _tc_prep(degpart, x, w1s):
    return pl.pallas_call(
        _tc_prep_body,
        grid=(NB,),
        in_specs=[
            pl.BlockSpec((6, NW, BLK), lambda b: (0, 0, b)),
            pl.BlockSpec((BLK, D), lambda b: (b, 0)),
            pl.BlockSpec((3, D, D), lambda b: (0, 0, 0)),
        ],
        out_specs=[
            pl.BlockSpec((6, BLK), lambda b: (0, b)),
            pl.BlockSpec((BLK, D), lambda b: (b, 0)),
            pl.BlockSpec((BLK, D), lambda b: (b, 0)),
            pl.BlockSpec((BLK, D), lambda b: (b, 0)),
        ],
        out_shape=[
            jax.ShapeDtypeStruct((6, NP), jnp.float32),
            jax.ShapeDtypeStruct((NP, D), jnp.float32),
            jax.ShapeDtypeStruct((NP, D), jnp.float32),
            jax.ShapeDtypeStruct((NP, D), jnp.float32),
        ],
    )(degpart, x, w1s)


@functools.partial(
    pl.kernel,
    out_type=[
        jax.ShapeDtypeStruct((NC, 3, NP, D), jnp.float32),
        jax.ShapeDtypeStruct((3 * NW * NP,), jnp.float32),
    ],
    mesh=_MESH,
    scratch_types=[
        pltpu.VMEM_SHARED((NP, D), jnp.float32),
        [pltpu.VMEM((CW, D), jnp.float32)] * 2,
        pltpu.VMEM((EPT,), jnp.int32),
        pltpu.VMEM((EPT,), jnp.int32),
        pltpu.VMEM((NP,), jnp.float32),
        pltpu.VMEM((NP,), jnp.float32),
        [pltpu.SemaphoreType.DMA] * 4,
    ],
    compiler_params=pltpu.CompilerParams(needs_layout_passes=False),
)
def _sc_aggregate(p0, p1, p2, nd0, nd1, nd2, s0, d0, s1, d1, s2, d2,
                  hpart, spart, acc, rows2, src_c, dst_c, nd_v, s_v, sems):
    cid = lax.axis_index("c")
    sid = lax.axis_index("s")
    wid = sid * NC + cid
    base = wid * EPT
    tail = E - (NW - 1) * EPT
    npairs = jnp.where(wid == NW - 1, tail // (2 * CW), NPAIR)
    rows0, rows1 = rows2
    gsem0, gsem1, ssem0, ssem1 = sems

    for r, (p_hbm, nd_hbm, src_hbm, dst_hbm) in enumerate(
            ((p0, nd0, s0, d0), (p1, nd1, s1, d1), (p2, nd2, s2, d2))):
        def zrow(i, _):
            rows0[i // (D // 16), pl.ds((i % (D // 16)) * 16, 16)] = (
                jnp.zeros((16,), jnp.float32))
            return 0
        lax.fori_loop(0, CW * (D // 16), zrow, 0)
        for k in range(STRIPE // CW):
            pltpu.sync_copy(rows0, acc.at[pl.ds(sid * STRIPE + k * CW, CW)])

        def zs(i, _):
            s_v[pl.ds(i * 16, 16)] = jnp.zeros((16,), jnp.float32)
            return 0
        lax.fori_loop(0, NP // 16, zs, 0)

        pltpu.sync_copy(nd_hbm, nd_v)

        @pl.when(wid < NW - 1)
        def _(src_hbm=src_hbm, dst_hbm=dst_hbm):
            pltpu.sync_copy(src_hbm.at[pl.ds(base, EPT)], src_c)
            pltpu.sync_copy(dst_hbm.at[pl.ds(base, EPT)], dst_c)

        @pl.when(wid == NW - 1)
        def _(src_hbm=src_hbm, dst_hbm=dst_hbm):
            pltpu.sync_copy(src_hbm.at[pl.ds(base, tail)],
                            src_c.at[pl.ds(0, tail)])
            pltpu.sync_copy(dst_hbm.at[pl.ds(base, tail)],
                            dst_c.at[pl.ds(0, tail)])

        plsc.subcore_barrier()

        def pbody(p, _, r=r, p_hbm=p_hbm, dst_hbm=dst_hbm):
            @pl.when(p > 0)
            def _(p_hbm=p_hbm):
                pltpu.make_async_copy(p_hbm.at[pl.ds(0, CW)], rows0,
                                      ssem0).wait()
                pltpu.make_async_copy(p_hbm.at[pl.ds(0, CW)], rows1,
                                      ssem1).wait()
            oa = p * (2 * CW)
            ga = pltpu.async_copy(p_hbm.at[src_c.at[pl.ds(oa, CW)]],
                                  rows0, gsem0)
            gb = pltpu.async_copy(p_hbm.at[src_c.at[pl.ds(oa + CW, CW)]],
                                  rows1, gsem1)
            for j in range(2 * CW // 16):
                sv = src_c[pl.ds(oa + j * 16, 16)]
                dv = dst_c[pl.ds(oa + j * 16, 16)]
                w = plsc.load_gather(nd_v, [dv])
                plsc.addupdate_scatter(s_v, [sv], w)
            ga.wait()
            pltpu.async_copy(rows0, acc.at[dst_c.at[pl.ds(oa, CW)]],
                             ssem0, add=True)
            gb.wait()
            pltpu.async_copy(rows1, acc.at[dst_c.at[pl.ds(oa + CW, CW)]],
                             ssem1, add=True)
            return 0
        lax.fori_loop(0, npairs, pbody, 0)
        pltpu.make_async_copy(p_hbm.at[pl.ds(0, CW)], rows0, ssem0).wait()
        pltpu.make_async_copy(p_hbm.at[pl.ds(0, CW)], rows1, ssem1).wait()

        pltpu.sync_copy(s_v, spart.at[pl.ds((r * NW + wid) * NP, NP)])

        plsc.subcore_barrier()

        for k in range(STRIPE // CW):
            off = sid * STRIPE + k * CW
            pltpu.sync_copy(acc.at[pl.ds(off, CW)], rows0)
            pltpu.sync_copy(rows0, hpart.at[cid].at[r].at[pl.ds(off, CW)])


def _tc_finish_body(hpart_ref, spart_ref, norms_ref, w2_ref, b1_ref, b2_ref,
                    wc_ref, bc_ref, out_ref, g_ref):
    b = pl.program_id(0)

    @pl.when(b == 0)
    def _():
        g_ref[...] = jnp.zeros_like(g_ref)

    hp = hpart_ref[...]
    b1sum = jnp.sum(b1_ref[...], axis=0)
    norm = norms_ref[...]
    pre = b1sum[None, :]
    for r in range(3):
        pre = pre + (hp[0, r] + hp[1, r]) * norm[2 * r + 1][:, None]
    h = jnp.maximum(pre, 0.0)
    s = jnp.sum(spart_ref[...], axis=1)
    ns = jnp.stack([norm[0], norm[2], norm[4]])
    c = ns * s
    g_ref[...] += jnp.dot(c, h, preferred_element_type=jnp.float32)

    @pl.when(b == NB - 1)
    def _():
        g = g_ref[...]
        acc = jnp.zeros((1, D), jnp.float32)
        for r in range(3):
            acc = acc + jnp.dot(g[r:r + 1], w2_ref[r],
                                preferred_element_type=jnp.float32)
        hg = acc / float(N) + jnp.sum(b2_ref[...], axis=0)[None, :]
        logits = jnp.dot(hg, wc_ref[...],
                         preferred_element_type=jnp.float32) + bc_ref[...]
        z = logits - jnp.max(logits, axis=-1, keepdims=True)
        ez = jnp.exp(z)
        out_ref[...] = ez / jnp.sum(ez, axis=-1, keepdims=True)


def _tc_finish(hpart, spart, norms, w2s, b1s, b2s, wc, bc2):
    return pl.pallas_call(
        _tc_finish_body,
        grid=(NB,),
        in_specs=[
            pl.BlockSpec((NC, 3, BLK, D), lambda b: (0, 0, b, 0)),
            pl.BlockSpec((3, NW, BLK), lambda b: (0, 0, b)),
            pl.BlockSpec((6, BLK), lambda b: (0, b)),
            pl.BlockSpec((3, D, D), lambda b: (0, 0, 0)),
            pl.BlockSpec((3, D), lambda b: (0, 0)),
            pl.BlockSpec((3, D), lambda b: (0, 0)),
            pl.BlockSpec((D, NCLS), lambda b: (0, 0)),
            pl.BlockSpec((1, NCLS), lambda b: (0, 0)),
        ],
        out_specs=pl.BlockSpec((1, NCLS), lambda b: (0, 0)),
        out_shape=jax.ShapeDtypeStruct((1, NCLS), jnp.float32),
        scratch_shapes=[pltpu.VMEM((3, D), jnp.float32)],
    )(hpart, spart, norms, w2s, b1s, b2s, wc, bc2)


def kernel(x, edge_index_rel0, edge_index_rel1, edge_index_rel2,
           W1_0, b1_0, W2_0, b2_0,
           W1_1, b1_1, W2_1, b2_1,
           W1_2, b1_2, W2_2, b2_2,
           Wc, bc):
    s0 = edge_index_rel0[0]
    d0 = edge_index_rel0[1]
    s1 = edge_index_rel1[0]
    d1 = edge_index_rel1[1]
    s2 = edge_index_rel2[0]
    d2 = edge_index_rel2[1]

    degpart = _sc_degrees(s0, d0, s1, d1, s2, d2).reshape(6, NW, NP)

    xp = jnp.pad(x, ((0, NP - N), (0, 0)))
    w1s = jnp.stack([W1_0, W1_1, W1_2])
    norms, p0, p1, p2 = _tc_prep(degpart, xp, w1s)

    hpart, spart = _sc_aggregate(p0, p1, p2, norms[1], norms[3], norms[5],
                                 s0, d0, s1, d1, s2, d2)
    spart = spart.reshape(3, NW, NP)

    w2s = jnp.stack([W2_0, W2_1, W2_2])
    b1s = jnp.stack([b1_0, b1_1, b1_2])
    b2s = jnp.stack([b2_0, b2_1, b2_2])
    out = _tc_finish(hpart, spart, norms, w2s, b1s, b2s, Wc,
                     bc.reshape(1, NCLS))
    return out.reshape(NCLS)


# speedup vs baseline: 1.0327x; 1.0327x over previous
import functools

import jax
import jax.numpy as jnp
from jax import lax
from jax.experimental import pallas as pl
from jax.experimental.pallas import tpu as pltpu
from jax.experimental.pallas import tpu_sc as plsc

N = 10000
NP = 10240
D = 128
NCLS = 10
E = 160000
NC = 2
NS = 16
NW = NC * NS
IW = 128
EROWS = E // IW
ROWS_PER_TILE = 40
EPT = ROWS_PER_TILE * IW
CW = 80
NPAIR = EPT // (2 * CW)
STRIPE = NP // NS
NB = NP // 512
BLK = 512

_MESH = plsc.VectorSubcoreMesh(
    core_axis_name="c", subcore_axis_name="s", num_cores=NC, num_subcores=NS)


def _wid():
    return lax.axis_index("s") * NC + lax.axis_index("c")


def _row_range(wid):
    start = wid * ROWS_PER_TILE
    end = jnp.minimum(start + ROWS_PER_TILE, EROWS)
    return start, end


@functools.partial(
    pl.kernel,
    out_type=jax.ShapeDtypeStruct((6 * NW * NP,), jnp.float32),
    mesh=_MESH,
    scratch_types=[
        [pltpu.VMEM((NP,), jnp.float32)] * 6,
        pltpu.VMEM((EPT,), jnp.int32),
    ],
    compiler_params=pltpu.CompilerParams(needs_layout_passes=False),
)
def _sc_degrees(s0, d0, s1, d1, s2, d2, degpart, deg_vs, idx_v):
    wid = _wid()
    ones = jnp.full((16,), 1.0, jnp.float32)
    base = wid * EPT
    ngroups = jnp.where(wid == NW - 1, (E - (NW - 1) * EPT) // 16, EPT // 16)

    for deg_v in deg_vs:
        def zbody(i, _, deg_v=deg_v):
            deg_v[pl.ds(i * 16, 16)] = jnp.zeros((16,), jnp.float32)
            return 0
        lax.fori_loop(0, NP // 16, zbody, 0)

    for a, idx_hbm in enumerate((s0, d0, s1, d1, s2, d2)):
        @pl.when(wid < NW - 1)
        def _(idx_hbm=idx_hbm):
            pltpu.sync_copy(idx_hbm.at[pl.ds(base, EPT)], idx_v)

        @pl.when(wid == NW - 1)
        def _(idx_hbm=idx_hbm):
            tail = E - (NW - 1) * EPT
            pltpu.sync_copy(idx_hbm.at[pl.ds(base, tail)],
                            idx_v.at[pl.ds(0, tail)])

        def gbody(g, _, a=a):
            iv = idx_v[pl.ds(g * 16, 16)]
            plsc.addupdate_scatter(deg_vs[a], [iv], ones)
            return 0
        lax.fori_loop(0, ngroups, gbody, 0)

    for a in range(6):
        pltpu.sync_copy(deg_vs[a], degpart.at[pl.ds((a * NW + wid) * NP, NP)])


def _tc_prep_body(degpart_ref, x_ref, w1_ref, n0_ref, p0_ref, p1_ref, p2_ref):
    deg = jnp.sum(degpart_ref[...], axis=1)
    norm = jnp.where(deg > 0.0, lax.rsqrt(deg), 0.0)
    n0_ref[...] = norm
    x_blk = x_ref[...]
    for r, p_ref in enumerate((p0_ref, p1_ref, p2_ref)):
        xw = jnp.dot(x_blk, w1_ref[r], preferred_element_type=jnp.float32)
        p_ref[...] = xw * norm[2 * r][:, None]


def _tc_prep(degpart, x, w1s):
    return pl.pallas_call(
        _tc_prep_body,
        grid=(NB,),
        in_specs=[
            pl.BlockSpec((6, NW, BLK), lambda b: (0, 0, b)),
            pl.BlockSpec((BLK, D), lambda b: (b, 0)),
            pl.BlockSpec((3, D, D), lambda b: (0, 0, 0)),
        ],
        out_specs=[
            pl.BlockSpec((6, BLK), lambda b: (0, b)),
            pl.BlockSpec((BLK, D), lambda b: (b, 0)),
            pl.BlockSpec((BLK, D), lambda b: (b, 0)),
            pl.BlockSpec((BLK, D), lambda b: (b, 0)),
        ],
        out_shape=[
            jax.ShapeDtypeStruct((6, NP), jnp.float32),
            jax.ShapeDtypeStruct((NP, D), jnp.float32),
            jax.ShapeDtypeStruct((NP, D), jnp.float32),
            jax.ShapeDtypeStruct((NP, D), jnp.float32),
        ],
    )(degpart, x, w1s)


@functools.partial(
    pl.kernel,
    out_type=[
        jax.ShapeDtypeStruct((NC, 3, NP, D), jnp.float32),
        jax.ShapeDtypeStruct((3 * NW * NP,), jnp.float32),
    ],
    mesh=_MESH,
    scratch_types=[
        pltpu.VMEM_SHARED((NP, D), jnp.float32),
        [pltpu.VMEM((CW, D), jnp.float32)] * 2,
        pltpu.VMEM((EPT,), jnp.int32),
        [pltpu.VMEM((CW,), jnp.int32)] * 2,
        pltpu.VMEM((NP,), jnp.float32),
        pltpu.VMEM((NP,), jnp.float32),
        [pltpu.SemaphoreType.DMA] * 4,
    ],
    compiler_params=pltpu.CompilerParams(needs_layout_passes=False),
)
def _sc_aggregate(p0, p1, p2, nd0, nd1, nd2, s0, d0, s1, d1, s2, d2,
                  hpart, spart, acc, rows2, src_c, dst2, nd_v, s_v, sems):
    cid = lax.axis_index("c")
    sid = lax.axis_index("s")
    wid = sid * NC + cid
    base = wid * EPT
    tail = E - (NW - 1) * EPT
    npairs = jnp.where(wid == NW - 1, tail // (2 * CW), NPAIR)
    rows0, rows1 = rows2
    dst_v0, dst_v1 = dst2
    gsem0, gsem1, ssem0, ssem1 = sems

    for r, (p_hbm, nd_hbm, src_hbm, dst_hbm) in enumerate(
            ((p0, nd0, s0, d0), (p1, nd1, s1, d1), (p2, nd2, s2, d2))):
        def zrow(i, _):
            rows0[i // (D // 16), pl.ds((i % (D // 16)) * 16, 16)] = (
                jnp.zeros((16,), jnp.float32))
            return 0
        lax.fori_loop(0, CW * (D // 16), zrow, 0)
        for k in range(STRIPE // CW):
            pltpu.sync_copy(rows0, acc.at[pl.ds(sid * STRIPE + k * CW, CW)])

        def zs(i, _):
            s_v[pl.ds(i * 16, 16)] = jnp.zeros((16,), jnp.float32)
            return 0
        lax.fori_loop(0, NP // 16, zs, 0)

        pltpu.sync_copy(nd_hbm, nd_v)

        @pl.when(wid < NW - 1)
        def _(src_hbm=src_hbm):
            pltpu.sync_copy(src_hbm.at[pl.ds(base, EPT)], src_c)

        @pl.when(wid == NW - 1)
        def _(src_hbm=src_hbm):
            pltpu.sync_copy(src_hbm.at[pl.ds(base, tail)],
                            src_c.at[pl.ds(0, tail)])

        plsc.subcore_barrier()

        def pbody(p, _, r=r, p_hbm=p_hbm, dst_hbm=dst_hbm):
            @pl.when(p > 0)
            def _(p_hbm=p_hbm):
                pltpu.make_async_copy(p_hbm.at[pl.ds(0, CW)], rows0,
                                      ssem0).wait()
                pltpu.make_async_copy(p_hbm.at[pl.ds(0, CW)], rows1,
                                      ssem1).wait()
            ea = base + p * (2 * CW)
            oa = p * (2 * CW)
            da = pltpu.async_copy(dst_hbm.at[pl.ds(ea, CW)], dst_v0, gsem0)
            ga = pltpu.async_copy(p_hbm.at[src_c.at[pl.ds(oa, CW)]],
                                  rows0, gsem0)
            db = pltpu.async_copy(dst_hbm.at[pl.ds(ea + CW, CW)], dst_v1, gsem1)
            gb = pltpu.async_copy(p_hbm.at[src_c.at[pl.ds(oa + CW, CW)]],
                                  rows1, gsem1)
            da.wait()
            for j in range(CW // 16):
                sv = src_c[pl.ds(oa + j * 16, 16)]
                dv = dst_v0[pl.ds(j * 16, 16)]
                w = plsc.load_gather(nd_v, [dv])
                plsc.addupdate_scatter(s_v, [sv], w)
            ga.wait()
            pltpu.async_copy(rows0, acc.at[dst_v0], ssem0, add=True)
            db.wait()
            for j in range(CW // 16):
                sv = src_c[pl.ds(oa + CW + j * 16, 16)]
                dv = dst_v1[pl.ds(j * 16, 16)]
                w = plsc.load_gather(nd_v, [dv])
                plsc.addupdate_scatter(s_v, [sv], w)
            gb.wait()
            pltpu.async_copy(rows1, acc.at[dst_v1], ssem1, add=True)
            return 0
        lax.fori_loop(0, npairs, pbody, 0)
        pltpu.make_async_copy(p_hbm.at[pl.ds(0, CW)], rows0, ssem0).wait()
        pltpu.make_async_copy(p_hbm.at[pl.ds(0, CW)], rows1, ssem1).wait()

        pltpu.sync_copy(s_v, spart.at[pl.ds((r * NW + wid) * NP, NP)])

        plsc.subcore_barrier()

        for k in range(STRIPE // CW):
            off = sid * STRIPE + k * CW
            pltpu.sync_copy(acc.at[pl.ds(off, CW)], rows0)
            pltpu.sync_copy(rows0, hpart.at[cid].at[r].at[pl.ds(off, CW)])


def _tc_finish_body(hpart_ref, spart_ref, norms_ref, w2_ref, b1_ref, b2_ref,
                    wc_ref, bc_ref, out_ref, g_ref):
    b = pl.program_id(0)

    @pl.when(b == 0)
    def _():
        g_ref[...] = jnp.zeros_like(g_ref)

    hp = hpart_ref[...]
    b1sum = jnp.sum(b1_ref[...], axis=0)
    norm = norms_ref[...]
    pre = b1sum[None, :]
    for r in range(3):
        pre = pre + (hp[0, r] + hp[1, r]) * norm[2 * r + 1][:, None]
    h = jnp.maximum(pre, 0.0)
    s = jnp.sum(spart_ref[...], axis=1)
    ns = jnp.stack([norm[0], norm[2], norm[4]])
    c = ns * s
    g_ref[...] += jnp.dot(c, h, preferred_element_type=jnp.float32)

    @pl.when(b == NB - 1)
    def _():
        g = g_ref[...]
        acc = jnp.zeros((1, D), jnp.float32)
        for r in range(3):
            acc = acc + jnp.dot(g[r:r + 1], w2_ref[r],
                                preferred_element_type=jnp.float32)
        hg = acc / float(N) + jnp.sum(b2_ref[...], axis=0)[None, :]
        logits = jnp.dot(hg, wc_ref[...],
                         preferred_element_type=jnp.float32) + bc_ref[...]
        z = logits - jnp.max(logits, axis=-1, keepdims=True)
        ez = jnp.exp(z)
        out_ref[...] = ez / jnp.sum(ez, axis=-1, keepdims=True)


def _tc_finish(hpart, spart, norms, w2s, b1s, b2s, wc, bc2):
    return pl.pallas_call(
        _tc_finish_body,
        grid=(NB,),
        in_specs=[
            pl.BlockSpec((NC, 3, BLK, D), lambda b: (0, 0, b, 0)),
            pl.BlockSpec((3, NW, BLK), lambda b: (0, 0, b)),
            pl.BlockSpec((6, BLK), lambda b: (0, b)),
            pl.BlockSpec((3, D, D), lambda b: (0, 0, 0)),
            pl.BlockSpec((3, D), lambda b: (0, 0)),
            pl.BlockSpec((3, D), lambda b: (0, 0)),
            pl.BlockSpec((D, NCLS), lambda b: (0, 0)),
            pl.BlockSpec((1, NCLS), lambda b: (0, 0)),
        ],
        out_specs=pl.BlockSpec((1, NCLS), lambda b: (0, 0)),
        out_shape=jax.ShapeDtypeStruct((1, NCLS), jnp.float32),
        scratch_shapes=[pltpu.VMEM((3, D), jnp.float32)],
    )(hpart, spart, norms, w2s, b1s, b2s, wc, bc2)


def kernel(x, edge_index_rel0, edge_index_rel1, edge_index_rel2,
           W1_0, b1_0, W2_0, b2_0,
           W1_1, b1_1, W2_1, b2_1,
           W1_2, b1_2, W2_2, b2_2,
           Wc, bc):
    s0 = edge_index_rel0[0]
    d0 = edge_index_rel0[1]
    s1 = edge_index_rel1[0]
    d1 = edge_index_rel1[1]
    s2 = edge_index_rel2[0]
    d2 = edge_index_rel2[1]

    degpart = _sc_degrees(s0, d0, s1, d1, s2, d2).reshape(6, NW, NP)

    xp = jnp.pad(x, ((0, NP - N), (0, 0)))
    w1s = jnp.stack([W1_0, W1_1, W1_2])
    norms, p0, p1, p2 = _tc_prep(degpart, xp, w1s)

    hpart, spart = _sc_aggregate(p0, p1, p2, norms[1], norms[3], norms[5],
                                 s0, d0, s1, d1, s2, d2)
    spart = spart.reshape(3, NW, NP)

    w2s = jnp.stack([W2_0, W2_1, W2_2])
    b1s = jnp.stack([b1_0, b1_1, b1_2])
    b2s = jnp.stack([b2_0, b2_1, b2_2])
    out = _tc_finish(hpart, spart, norms, w2s, b1s, b2s, Wc,
                     bc.reshape(1, NCLS))
    return out.reshape(NCLS)
